# baseline (speedup 1.0000x reference)
import jax
import jax.numpy as jnp
import numpy as np
from jax.experimental import pallas as pl
from jax.experimental.pallas import tpu as pltpu

HID = 32
FEA = 64
NB = 256
HB = 128
G4 = 4 * HID
T_BLK = 64
UNROLL = 64


def _cell(x_t, h, c, wx2, wh3, bias3):
    xg = jnp.dot(x_t, wx2, preferred_element_type=jnp.float32)
    xgb = xg + bias3[:, 0:2 * G4]
    xg2 = pltpu.roll(xgb[:, G4:2 * G4], 3 * HID, 1)
    hh = jnp.dot(h, wh3, preferred_element_type=jnp.float32)
    a0 = jnp.tanh(xgb[:, 0:G4] + hh[:, 0:G4])
    ai = jnp.tanh(xgb[:, G4:G4 + HID] + hh[:, G4:G4 + HID])
    ag = jnp.tanh(xg2[:, 0:HID] + hh[:, 2 * G4:2 * G4 + HID])
    c = (0.5 * a0[:, 0:HID] + 0.5) * c + (0.5 * ai + 0.5) * ag
    ro = pltpu.roll(a0, HID, 1)
    h = (0.5 * ro[:, 0:HID] + 0.5) * jnp.tanh(c)
    return h, c


def _lstm_kernel(x_ref, wx3_ref, wh3_ref, b3_ref, out_ref, c_ref, h_ref):
    tb = pl.program_id(0)

    @pl.when(tb == 0)
    def _():
        c_ref[...] = jnp.zeros_like(c_ref)
        h_ref[...] = jnp.zeros_like(h_ref)

    wx3 = wx3_ref[...]
    wh3 = wh3_ref[...]
    bias3 = b3_ref[...]

    def body(k, carry_token):
        t0 = k * UNROLL
        c = c_ref[...]
        h = h_ref[...]
        for j in range(UNROLL):
            t = t0 + j
            h, c = _cell(x_ref[t], h, c, wx3, wh3, bias3)
            out_ref[t] = h
        c_ref[...] = c
        h_ref[...] = h
        return carry_token

    jax.lax.fori_loop(0, T_BLK // UNROLL, body, 0)


def kernel(x, Wx, Wh, b):
    B, T, F = x.shape
    xT = jnp.swapaxes(x.astype(jnp.bfloat16), 0, 1)
    perm = np.concatenate([np.arange(HID, 2 * HID), np.arange(0, HID),
                           np.arange(2 * HID, 4 * HID)])
    gscale = np.concatenate([np.full(2 * HID, 0.5), np.ones(HID),
                             np.full(HID, 0.5)]).astype(np.float32)
    s32 = (np.arange(G4) + HID) % G4
    s64 = (np.arange(G4) + 2 * HID) % G4
    wxp = Wx[:, perm] * gscale
    whp = Wh[:, perm] * gscale
    bp = b[perm] * gscale
    wx2 = jnp.concatenate([wxp, wxp[:, s32]],
                          axis=1).astype(jnp.bfloat16)
    wh3 = jnp.concatenate([whp, whp[:, s32], whp[:, s64]], axis=1)
    b3 = jnp.concatenate([bp, bp[s32], bp[s64]]).reshape(1, 3 * G4)

    ysT = pl.pallas_call(
        _lstm_kernel,
        out_shape=jax.ShapeDtypeStruct((T, B, HID), x.dtype),
        grid=(T // T_BLK,),
        in_specs=[
            pl.BlockSpec((T_BLK, NB, FEA), lambda t: (t, 0, 0)),
            pl.BlockSpec((FEA, 2 * G4), lambda t: (0, 0)),
            pl.BlockSpec((HID, 3 * G4), lambda t: (0, 0)),
            pl.BlockSpec((1, 3 * G4), lambda t: (0, 0)),
        ],
        out_specs=pl.BlockSpec((T_BLK, NB, HID), lambda t: (t, 0, 0)),
        scratch_shapes=[
            pltpu.VMEM((NB, HID), jnp.float32),
            pltpu.VMEM((NB, HID), jnp.float32),
        ],
        compiler_params=pltpu.CompilerParams(
            dimension_semantics=("arbitrary",),
            vmem_limit_bytes=50 * 1024 * 1024,
        ),
        name="financial_rnn_lstm",
    )(xT, wx2, wh3, b3)
    return jnp.swapaxes(ysT, 0, 1)


# speedup vs baseline: 1.0407x; 1.0407x over previous
import jax
import jax.numpy as jnp
import numpy as np
from jax.experimental import pallas as pl
from jax.experimental.pallas import tpu as pltpu

HID = 32
FEA = 64
NB = 256
HB = 128
G4 = 4 * HID
T_BLK = 64
UNROLL = 64


def _cell(x_t, h, c, wx2, wh3, bias3):
    xg = jnp.dot(x_t, wx2, preferred_element_type=jnp.float32)
    xgb = xg + bias3[:, 0:2 * G4]
    xg2 = pltpu.roll(xgb[:, G4:2 * G4], 3 * HID, 1)
    hh = jnp.dot(h, wh3, preferred_element_type=jnp.float32)
    a0 = jnp.tanh(xgb[:, 0:G4] + hh[:, 0:G4])
    ai = jnp.tanh(xgb[:, G4:G4 + HID] + hh[:, G4:G4 + HID])
    ag = jnp.tanh(xg2[:, 0:HID] + hh[:, 2 * G4:2 * G4 + HID])
    c = (0.5 * a0[:, 0:HID] + 0.5) * c + (0.5 * ai + 0.5) * ag
    ro = pltpu.roll(a0, HID, 1)
    h = (0.5 * ro[:, 0:HID] + 0.5) * jnp.tanh(c)
    return h, c


def _lstm_kernel(x_ref, wx3_ref, wh3_ref, b3_ref, out_ref, c_ref, h_ref):
    tb = pl.program_id(0)

    @pl.when(tb == 0)
    def _():
        c_ref[...] = jnp.zeros_like(c_ref)
        h_ref[...] = jnp.zeros_like(h_ref)

    wx3 = wx3_ref[...]
    wh3 = wh3_ref[...]
    bias3 = b3_ref[...]

    def body(k, carry_token):
        t0 = k * UNROLL
        c = c_ref[...]
        h = h_ref[...]
        for j in range(UNROLL):
            t = t0 + j
            h, c = _cell(x_ref[t], h, c, wx3, wh3, bias3)
            out_ref[t] = h
        c_ref[...] = c
        h_ref[...] = h
        return carry_token

    jax.lax.fori_loop(0, T_BLK // UNROLL, body, 0)


def kernel(x, Wx, Wh, b):
    B, T, F = x.shape
    xT = jnp.swapaxes(x, 0, 1)
    perm = np.concatenate([np.arange(HID, 2 * HID), np.arange(0, HID),
                           np.arange(2 * HID, 4 * HID)])
    gscale = np.concatenate([np.full(2 * HID, 0.5), np.ones(HID),
                             np.full(HID, 0.5)]).astype(np.float32)
    s32 = (np.arange(G4) + HID) % G4
    s64 = (np.arange(G4) + 2 * HID) % G4
    wxp = Wx[:, perm] * gscale
    whp = Wh[:, perm] * gscale
    bp = b[perm] * gscale
    wx2 = jnp.concatenate([wxp, wxp[:, s32]],
                          axis=1)
    wh3 = jnp.concatenate([whp, whp[:, s32], whp[:, s64]], axis=1)
    b3 = jnp.concatenate([bp, bp[s32], bp[s64]]).reshape(1, 3 * G4)

    ysT = pl.pallas_call(
        _lstm_kernel,
        out_shape=jax.ShapeDtypeStruct((T, B, HID), x.dtype),
        grid=(T // T_BLK,),
        in_specs=[
            pl.BlockSpec((T_BLK, NB, FEA), lambda t: (t, 0, 0)),
            pl.BlockSpec((FEA, 2 * G4), lambda t: (0, 0)),
            pl.BlockSpec((HID, 3 * G4), lambda t: (0, 0)),
            pl.BlockSpec((1, 3 * G4), lambda t: (0, 0)),
        ],
        out_specs=pl.BlockSpec((T_BLK, NB, HID), lambda t: (t, 0, 0)),
        scratch_shapes=[
            pltpu.VMEM((NB, HID), jnp.float32),
            pltpu.VMEM((NB, HID), jnp.float32),
        ],
        compiler_params=pltpu.CompilerParams(
            dimension_semantics=("arbitrary",),
            vmem_limit_bytes=50 * 1024 * 1024,
        ),
        name="financial_rnn_lstm",
    )(xT, wx2, wh3, b3)
    return jnp.swapaxes(ysT, 0, 1)


# speedup vs baseline: 1.2365x; 1.1881x over previous
import jax
import jax.numpy as jnp
import numpy as np
from jax.experimental import pallas as pl
from jax.experimental.pallas import tpu as pltpu

HID = 32
FEA = 64
NB = 256
HB = 128
G4 = 4 * HID
T_BLK = 64
UNROLL = 64


def _cell(x_t, h, c, wx2, wh4, bias3):
    xg = jnp.dot(x_t, wx2, preferred_element_type=jnp.float32)
    xgb = xg + bias3[:, 0:2 * G4]
    xg2 = pltpu.roll(xgb[:, G4:2 * G4], 3 * HID, 1)
    xg3 = pltpu.roll(xgb[:, 0:G4], HID, 1)
    hh = jnp.dot(h, wh4, preferred_element_type=jnp.float32)
    af = jnp.tanh(xgb[:, 0:HID] + hh[:, 0:HID])
    ai = jnp.tanh(xgb[:, G4:G4 + HID] + hh[:, G4:G4 + HID])
    ag = jnp.tanh(xg2[:, 0:HID] + hh[:, 2 * G4:2 * G4 + HID])
    ao = jnp.tanh(xg3[:, 0:HID] + hh[:, 3 * G4:3 * G4 + HID])
    c = (0.5 * af + 0.5) * c + (0.5 * ai + 0.5) * ag
    h = (0.5 * ao + 0.5) * jnp.tanh(c)
    return h, c


def _lstm_kernel(x_ref, wx3_ref, wh3_ref, b3_ref, out_ref, c_ref, h_ref):
    tb = pl.program_id(0)

    @pl.when(tb == 0)
    def _():
        c_ref[...] = jnp.zeros_like(c_ref)
        h_ref[...] = jnp.zeros_like(h_ref)

    wx3 = wx3_ref[...]
    wh3 = wh3_ref[...]
    bias3 = b3_ref[...]

    def body(k, carry_token):
        t0 = k * UNROLL
        c = c_ref[...]
        h = h_ref[...]
        for j in range(UNROLL):
            t = t0 + j
            h, c = _cell(x_ref[t], h, c, wx3, wh3, bias3)
            out_ref[t] = h
        c_ref[...] = c
        h_ref[...] = h
        return carry_token

    jax.lax.fori_loop(0, T_BLK // UNROLL, body, 0)


def kernel(x, Wx, Wh, b):
    B, T, F = x.shape
    xT = jnp.swapaxes(x, 0, 1)
    perm = np.concatenate([np.arange(HID, 2 * HID), np.arange(0, HID),
                           np.arange(2 * HID, 4 * HID)])
    gscale = np.concatenate([np.full(2 * HID, 0.5), np.ones(HID),
                             np.full(HID, 0.5)]).astype(np.float32)
    s32 = (np.arange(G4) + HID) % G4
    s64 = (np.arange(G4) + 2 * HID) % G4
    wxp = Wx[:, perm] * gscale
    whp = Wh[:, perm] * gscale
    bp = b[perm] * gscale
    wx2 = jnp.concatenate([wxp, wxp[:, s32]],
                          axis=1)
    s96 = (np.arange(G4) + 3 * HID) % G4
    wh4 = jnp.concatenate([whp, whp[:, s32], whp[:, s64], whp[:, s96]],
                          axis=1)
    b3 = jnp.concatenate([bp, bp[s32], bp[s64]]).reshape(1, 3 * G4)

    ysT = pl.pallas_call(
        _lstm_kernel,
        out_shape=jax.ShapeDtypeStruct((T, B, HID), x.dtype),
        grid=(T // T_BLK,),
        in_specs=[
            pl.BlockSpec((T_BLK, NB, FEA), lambda t: (t, 0, 0)),
            pl.BlockSpec((FEA, 2 * G4), lambda t: (0, 0)),
            pl.BlockSpec((HID, 4 * G4), lambda t: (0, 0)),
            pl.BlockSpec((1, 3 * G4), lambda t: (0, 0)),
        ],
        out_specs=pl.BlockSpec((T_BLK, NB, HID), lambda t: (t, 0, 0)),
        scratch_shapes=[
            pltpu.VMEM((NB, HID), jnp.float32),
            pltpu.VMEM((NB, HID), jnp.float32),
        ],
        compiler_params=pltpu.CompilerParams(
            dimension_semantics=("arbitrary",),
            vmem_limit_bytes=50 * 1024 * 1024,
        ),
        name="financial_rnn_lstm",
    )(xT, wx2, wh4, b3)
    return jnp.swapaxes(ysT, 0, 1)
